# baseline (device time: 18303 ns/iter reference)
import jax
import jax.numpy as jnp
from jax import lax
from jax.experimental import pallas as pl
from jax.experimental.pallas import tpu as pltpu

N_DEV = 4
N_SEG = 4


def kernel(x, w_mat):
    m_per, k = x.shape
    _, n_per = w_mat.shape
    seg = m_per // N_SEG
    half = m_per // 2

    def body(x_ref, w_ref, out_ref,
             x_vmem, w_vmem, own_buf, from_left, from_right, diag, out_stage,
             in_sems, out_sems,
             s_r1r, s_r1l, r_r1r, r_r1l,
             s_r2r, s_r2l, r_r2r, r_r2l):
        my_pos = lax.axis_index("i")
        left = (my_pos - 1) % N_DEV
        right = (my_pos + 1) % N_DEV

        x_dma = pltpu.make_async_copy(x_ref, x_vmem, in_sems.at[0])
        w_dma = pltpu.make_async_copy(w_ref, w_vmem, in_sems.at[1])
        x_dma.start()
        w_dma.start()
        x_dma.wait()
        xb = x_vmem[...].astype(jnp.bfloat16)
        own_buf[...] = xb

        barrier_sem = pltpu.get_barrier_semaphore()
        for nbr in [left, right]:
            pl.semaphore_signal(
                barrier_sem, inc=1,
                device_id=(nbr,), device_id_type=pl.DeviceIdType.MESH,
            )
        pl.semaphore_wait(barrier_sem, 2)

        def seg_copy(src, dst, s, send_sem, recv_sem, idx, dev):
            return pltpu.make_async_remote_copy(
                src_ref=src.at[pl.ds(s * seg, seg)],
                dst_ref=dst.at[pl.ds(s * seg, seg)],
                send_sem=send_sem.at[idx], recv_sem=recv_sem.at[idx],
                device_id=(dev,), device_id_type=pl.DeviceIdType.MESH,
            )

        r1r = [seg_copy(own_buf, from_left, s, s_r1r, r_r1r, s, right)
               for s in range(N_SEG)]
        r1l = [seg_copy(own_buf, from_right, s, s_r1l, r_r1l, s, left)
               for s in range(N_SEG)]
        for s in range(N_SEG):
            r1r[s].start()
            r1l[N_SEG - 1 - s].start()

        w_dma.wait()
        w = w_vmem[...].astype(jnp.bfloat16)
        acc = jnp.dot(xb, w, preferred_element_type=jnp.float32)
        out_stage[pl.ds(my_pos * m_per, m_per), :] = jnp.maximum(acc, 0.0)
        own_out = pltpu.make_async_copy(
            out_stage.at[pl.ds(my_pos * m_per, m_per)],
            out_ref.at[pl.ds(my_pos * m_per, m_per)],
            out_sems.at[0],
        )
        own_out.start()

        r2r = [seg_copy(from_left, diag, j, s_r2r, r_r2r, j, right)
               for j in range(2)]
        r2l = [seg_copy(from_right, diag, s, s_r2l, r_r2l, j, left)
               for j, s in enumerate([N_SEG - 1, N_SEG - 2])]

        r1r[0].wait_recv()
        r2r[0].start()
        r1l[N_SEG - 1].wait_recv()
        r2l[0].start()
        r1r[1].wait_recv()
        r2r[1].start()
        r1l[N_SEG - 2].wait_recv()
        r2l[1].start()

        r1r[2].wait_recv()
        r1r[3].wait_recv()
        acc = jnp.dot(from_left[...], w, preferred_element_type=jnp.float32)
        out_stage[pl.ds(left * m_per, m_per), :] = jnp.maximum(acc, 0.0)
        left_out = pltpu.make_async_copy(
            out_stage.at[pl.ds(left * m_per, m_per)],
            out_ref.at[pl.ds(left * m_per, m_per)],
            out_sems.at[1],
        )
        left_out.start()

        r1l[1].wait_recv()
        r1l[0].wait_recv()
        acc = jnp.dot(from_right[...], w, preferred_element_type=jnp.float32)
        out_stage[pl.ds(right * m_per, m_per), :] = jnp.maximum(acc, 0.0)
        right_out = pltpu.make_async_copy(
            out_stage.at[pl.ds(right * m_per, m_per)],
            out_ref.at[pl.ds(right * m_per, m_per)],
            out_sems.at[2],
        )
        right_out.start()

        diag_pos = (my_pos + 2) % N_DEV
        r2r[0].wait_recv()
        r2r[1].wait_recv()
        acc = jnp.dot(diag[pl.ds(0, half)], w, preferred_element_type=jnp.float32)
        out_stage[pl.ds(diag_pos * m_per, half), :] = jnp.maximum(acc, 0.0)
        diag_out_top = pltpu.make_async_copy(
            out_stage.at[pl.ds(diag_pos * m_per, half)],
            out_ref.at[pl.ds(diag_pos * m_per, half)],
            out_sems.at[3],
        )
        diag_out_top.start()

        r2l[0].wait_recv()
        r2l[1].wait_recv()
        acc = jnp.dot(diag[pl.ds(half, half)], w, preferred_element_type=jnp.float32)
        out_stage[pl.ds(diag_pos * m_per + half, half), :] = jnp.maximum(acc, 0.0)
        diag_out_bot = pltpu.make_async_copy(
            out_stage.at[pl.ds(diag_pos * m_per + half, half)],
            out_ref.at[pl.ds(diag_pos * m_per + half, half)],
            out_sems.at[4],
        )
        diag_out_bot.start()

        for d in r1r + r1l + r2r + r2l:
            d.wait_send()
        for d in [own_out, left_out, right_out, diag_out_top, diag_out_bot]:
            d.wait()

    return pl.pallas_call(
        body,
        out_shape=jax.ShapeDtypeStruct((N_DEV * m_per, n_per), jnp.float32),
        in_specs=[
            pl.BlockSpec(memory_space=pltpu.MemorySpace.HBM),
            pl.BlockSpec(memory_space=pltpu.MemorySpace.HBM),
        ],
        out_specs=pl.BlockSpec(memory_space=pltpu.MemorySpace.HBM),
        scratch_shapes=[
            pltpu.VMEM((m_per, k), jnp.float32),
            pltpu.VMEM((k, n_per), jnp.float32),
            pltpu.VMEM((m_per, k), jnp.bfloat16),
            pltpu.VMEM((m_per, k), jnp.bfloat16),
            pltpu.VMEM((m_per, k), jnp.bfloat16),
            pltpu.VMEM((m_per, k), jnp.bfloat16),
            pltpu.VMEM((N_DEV * m_per, n_per), jnp.float32),
            pltpu.SemaphoreType.DMA((2,)),
            pltpu.SemaphoreType.DMA((5,)),
            pltpu.SemaphoreType.DMA((N_SEG,)),
            pltpu.SemaphoreType.DMA((N_SEG,)),
            pltpu.SemaphoreType.DMA((N_SEG,)),
            pltpu.SemaphoreType.DMA((N_SEG,)),
            pltpu.SemaphoreType.DMA((2,)),
            pltpu.SemaphoreType.DMA((2,)),
            pltpu.SemaphoreType.DMA((2,)),
            pltpu.SemaphoreType.DMA((2,)),
        ],
        compiler_params=pltpu.CompilerParams(collective_id=0),
    )(x, w_mat)


# device time: 17724 ns/iter; 1.0327x vs baseline; 1.0327x over previous
import jax
import jax.numpy as jnp
from jax import lax
from jax.experimental import pallas as pl
from jax.experimental.pallas import tpu as pltpu

N_DEV = 4
N_SEG = 4


def kernel(x, w_mat):
    m_per, k = x.shape
    _, n_per = w_mat.shape
    seg = m_per // N_SEG
    half = m_per // 2

    def body(x_ref, w_ref, out_ref,
             own_buf, from_left, from_right, diag,
             s_r1r, s_r1l, r_r1r, r_r1l,
             s_r2r, s_r2l, r_r2r, r_r2l):
        my_pos = lax.axis_index("i")
        left = (my_pos - 1) % N_DEV
        right = (my_pos + 1) % N_DEV

        xb = x_ref[...].astype(jnp.bfloat16)
        own_buf[...] = xb

        barrier_sem = pltpu.get_barrier_semaphore()
        for nbr in [left, right]:
            pl.semaphore_signal(
                barrier_sem, inc=1,
                device_id=(nbr,), device_id_type=pl.DeviceIdType.MESH,
            )
        pl.semaphore_wait(barrier_sem, 2)

        def seg_copy(src, dst, s, send_sem, recv_sem, idx, dev):
            return pltpu.make_async_remote_copy(
                src_ref=src.at[pl.ds(s * seg, seg)],
                dst_ref=dst.at[pl.ds(s * seg, seg)],
                send_sem=send_sem.at[idx], recv_sem=recv_sem.at[idx],
                device_id=(dev,), device_id_type=pl.DeviceIdType.MESH,
            )

        r1r = [seg_copy(own_buf, from_left, s, s_r1r, r_r1r, s, right)
               for s in range(N_SEG)]
        r1l = [seg_copy(own_buf, from_right, s, s_r1l, r_r1l, s, left)
               for s in range(N_SEG)]
        for s in range(N_SEG):
            r1r[s].start()
            r1l[N_SEG - 1 - s].start()

        w = w_ref[...].astype(jnp.bfloat16)
        acc = jnp.dot(xb, w, preferred_element_type=jnp.float32)
        out_ref[pl.ds(my_pos * m_per, m_per), :] = jnp.maximum(acc, 0.0)

        r2r = [seg_copy(from_left, diag, j, s_r2r, r_r2r, j, right)
               for j in range(2)]
        r2l = [seg_copy(from_right, diag, s, s_r2l, r_r2l, j, left)
               for j, s in enumerate([N_SEG - 1, N_SEG - 2])]

        r1r[0].wait_recv()
        r2r[0].start()
        r1l[N_SEG - 1].wait_recv()
        r2l[0].start()
        r1r[1].wait_recv()
        r2r[1].start()
        r1l[N_SEG - 2].wait_recv()
        r2l[1].start()

        r1r[2].wait_recv()
        r1r[3].wait_recv()
        acc = jnp.dot(from_left[...], w, preferred_element_type=jnp.float32)
        out_ref[pl.ds(left * m_per, m_per), :] = jnp.maximum(acc, 0.0)

        r1l[1].wait_recv()
        r1l[0].wait_recv()
        acc = jnp.dot(from_right[...], w, preferred_element_type=jnp.float32)
        out_ref[pl.ds(right * m_per, m_per), :] = jnp.maximum(acc, 0.0)

        diag_pos = (my_pos + 2) % N_DEV
        r2r[0].wait_recv()
        r2r[1].wait_recv()
        acc = jnp.dot(diag[pl.ds(0, half)], w, preferred_element_type=jnp.float32)
        out_ref[pl.ds(diag_pos * m_per, half), :] = jnp.maximum(acc, 0.0)

        r2l[0].wait_recv()
        r2l[1].wait_recv()
        acc = jnp.dot(diag[pl.ds(half, half)], w, preferred_element_type=jnp.float32)
        out_ref[pl.ds(diag_pos * m_per + half, half), :] = jnp.maximum(acc, 0.0)

        for d in r1r + r1l + r2r + r2l:
            d.wait_send()

    return pl.pallas_call(
        body,
        out_shape=jax.ShapeDtypeStruct((N_DEV * m_per, n_per), jnp.float32),
        in_specs=[
            pl.BlockSpec(memory_space=pltpu.VMEM),
            pl.BlockSpec(memory_space=pltpu.VMEM),
        ],
        out_specs=pl.BlockSpec(memory_space=pltpu.VMEM),
        scratch_shapes=[
            pltpu.VMEM((m_per, k), jnp.bfloat16),
            pltpu.VMEM((m_per, k), jnp.bfloat16),
            pltpu.VMEM((m_per, k), jnp.bfloat16),
            pltpu.VMEM((m_per, k), jnp.bfloat16),
            pltpu.SemaphoreType.DMA((N_SEG,)),
            pltpu.SemaphoreType.DMA((N_SEG,)),
            pltpu.SemaphoreType.DMA((N_SEG,)),
            pltpu.SemaphoreType.DMA((N_SEG,)),
            pltpu.SemaphoreType.DMA((2,)),
            pltpu.SemaphoreType.DMA((2,)),
            pltpu.SemaphoreType.DMA((2,)),
            pltpu.SemaphoreType.DMA((2,)),
        ],
        compiler_params=pltpu.CompilerParams(collective_id=0),
    )(x, w_mat)


# device time: 16793 ns/iter; 1.0899x vs baseline; 1.0554x over previous
import jax
import jax.numpy as jnp
from jax import lax
from jax.experimental import pallas as pl
from jax.experimental.pallas import tpu as pltpu

N_DEV = 4
N_SEG = 4


def kernel(x, w_mat):
    x = x.astype(jnp.bfloat16)
    w_mat = w_mat.astype(jnp.bfloat16)
    m_per, k = x.shape
    _, n_per = w_mat.shape
    seg = m_per // N_SEG
    half = m_per // 2

    def body(x_ref, w_ref, out_ref,
             from_left, from_right, diag,
             s_r1r, s_r1l, r_r1r, r_r1l,
             s_r2r, s_r2l, r_r2r, r_r2l):
        my_pos = lax.axis_index("i")
        left = (my_pos - 1) % N_DEV
        right = (my_pos + 1) % N_DEV

        barrier_sem = pltpu.get_barrier_semaphore()
        for nbr in [left, right]:
            pl.semaphore_signal(
                barrier_sem, inc=1,
                device_id=(nbr,), device_id_type=pl.DeviceIdType.MESH,
            )
        pl.semaphore_wait(barrier_sem, 2)

        def seg_copy(src, dst, s, send_sem, recv_sem, idx, dev):
            return pltpu.make_async_remote_copy(
                src_ref=src.at[pl.ds(s * seg, seg)],
                dst_ref=dst.at[pl.ds(s * seg, seg)],
                send_sem=send_sem.at[idx], recv_sem=recv_sem.at[idx],
                device_id=(dev,), device_id_type=pl.DeviceIdType.MESH,
            )

        r1r = [seg_copy(x_ref, from_left, s, s_r1r, r_r1r, s, right)
               for s in range(N_SEG)]
        r1l = [seg_copy(x_ref, from_right, s, s_r1l, r_r1l, s, left)
               for s in range(N_SEG)]
        for s in range(N_SEG):
            r1r[s].start()
            r1l[N_SEG - 1 - s].start()

        w = w_ref[...]
        acc = jnp.dot(x_ref[...], w, preferred_element_type=jnp.float32)
        out_ref[pl.ds(my_pos * m_per, m_per), :] = jnp.maximum(acc, 0.0)

        r2r = [seg_copy(from_left, diag, j, s_r2r, r_r2r, j, right)
               for j in range(2)]
        r2l = [seg_copy(from_right, diag, s, s_r2l, r_r2l, j, left)
               for j, s in enumerate([N_SEG - 1, N_SEG - 2])]

        r1r[0].wait_recv()
        r2r[0].start()
        r1l[N_SEG - 1].wait_recv()
        r2l[0].start()
        r1r[1].wait_recv()
        r2r[1].start()
        r1l[N_SEG - 2].wait_recv()
        r2l[1].start()

        r1r[2].wait_recv()
        r1r[3].wait_recv()
        acc = jnp.dot(from_left[...], w, preferred_element_type=jnp.float32)
        out_ref[pl.ds(left * m_per, m_per), :] = jnp.maximum(acc, 0.0)

        r1l[1].wait_recv()
        r1l[0].wait_recv()
        acc = jnp.dot(from_right[...], w, preferred_element_type=jnp.float32)
        out_ref[pl.ds(right * m_per, m_per), :] = jnp.maximum(acc, 0.0)

        diag_pos = (my_pos + 2) % N_DEV
        r2r[0].wait_recv()
        r2r[1].wait_recv()
        acc = jnp.dot(diag[pl.ds(0, half)], w, preferred_element_type=jnp.float32)
        out_ref[pl.ds(diag_pos * m_per, half), :] = jnp.maximum(acc, 0.0)

        r2l[0].wait_recv()
        r2l[1].wait_recv()
        acc = jnp.dot(diag[pl.ds(half, half)], w, preferred_element_type=jnp.float32)
        out_ref[pl.ds(diag_pos * m_per + half, half), :] = jnp.maximum(acc, 0.0)

        for d in r1r + r1l + r2r + r2l:
            d.wait_send()

    return pl.pallas_call(
        body,
        out_shape=jax.ShapeDtypeStruct((N_DEV * m_per, n_per), jnp.float32),
        in_specs=[
            pl.BlockSpec(memory_space=pltpu.VMEM),
            pl.BlockSpec(memory_space=pltpu.VMEM),
        ],
        out_specs=pl.BlockSpec(memory_space=pltpu.VMEM),
        scratch_shapes=[
            pltpu.VMEM((m_per, k), jnp.bfloat16),
            pltpu.VMEM((m_per, k), jnp.bfloat16),
            pltpu.VMEM((m_per, k), jnp.bfloat16),
            pltpu.SemaphoreType.DMA((N_SEG,)),
            pltpu.SemaphoreType.DMA((N_SEG,)),
            pltpu.SemaphoreType.DMA((N_SEG,)),
            pltpu.SemaphoreType.DMA((N_SEG,)),
            pltpu.SemaphoreType.DMA((2,)),
            pltpu.SemaphoreType.DMA((2,)),
            pltpu.SemaphoreType.DMA((2,)),
            pltpu.SemaphoreType.DMA((2,)),
        ],
        compiler_params=pltpu.CompilerParams(collective_id=0),
    )(x, w_mat)


# device time: 16770 ns/iter; 1.0914x vs baseline; 1.0014x over previous
import jax
import jax.numpy as jnp
from jax import lax
from jax.experimental import pallas as pl
from jax.experimental.pallas import tpu as pltpu

N_DEV = 4
N_SEG = 4


def kernel(x, w_mat):
    x = lax.optimization_barrier(x.astype(jnp.bfloat16))
    w_mat = w_mat.astype(jnp.bfloat16)
    m_per, k = x.shape
    _, n_per = w_mat.shape
    seg = m_per // N_SEG
    half = m_per // 2

    def body(x_ref, w_ref, out_ref,
             from_left, from_right, diag,
             s_r1r, s_r1l, r_r1r, r_r1l,
             s_r2r, s_r2l, r_r2r, r_r2l):
        my_pos = lax.axis_index("i")
        left = (my_pos - 1) % N_DEV
        right = (my_pos + 1) % N_DEV

        barrier_sem = pltpu.get_barrier_semaphore()
        for nbr in [left, right]:
            pl.semaphore_signal(
                barrier_sem, inc=1,
                device_id=(nbr,), device_id_type=pl.DeviceIdType.MESH,
            )
        pl.semaphore_wait(barrier_sem, 2)

        def seg_copy(src, dst, s, send_sem, recv_sem, idx, dev):
            return pltpu.make_async_remote_copy(
                src_ref=src.at[pl.ds(s * seg, seg)],
                dst_ref=dst.at[pl.ds(s * seg, seg)],
                send_sem=send_sem.at[idx], recv_sem=recv_sem.at[idx],
                device_id=(dev,), device_id_type=pl.DeviceIdType.MESH,
            )

        r1r = [seg_copy(x_ref, from_left, s, s_r1r, r_r1r, s, right)
               for s in range(N_SEG)]
        r1l = [seg_copy(x_ref, from_right, s, s_r1l, r_r1l, s, left)
               for s in range(N_SEG)]
        for s in range(N_SEG):
            r1r[s].start()
            r1l[N_SEG - 1 - s].start()

        w = w_ref[...]
        acc = jnp.dot(x_ref[...], w, preferred_element_type=jnp.float32)
        out_ref[pl.ds(my_pos * m_per, m_per), :] = jnp.maximum(acc, 0.0)

        r2r = [seg_copy(from_left, diag, j, s_r2r, r_r2r, j, right)
               for j in range(2)]
        r2l = [seg_copy(from_right, diag, s, s_r2l, r_r2l, j, left)
               for j, s in enumerate([N_SEG - 1, N_SEG - 2])]

        r1r[0].wait_recv()
        r2r[0].start()
        r1l[N_SEG - 1].wait_recv()
        r2l[0].start()
        r1r[1].wait_recv()
        r2r[1].start()
        r1l[N_SEG - 2].wait_recv()
        r2l[1].start()

        r1r[2].wait_recv()
        r1r[3].wait_recv()
        acc = jnp.dot(from_left[...], w, preferred_element_type=jnp.float32)
        out_ref[pl.ds(left * m_per, m_per), :] = jnp.maximum(acc, 0.0)

        r1l[1].wait_recv()
        r1l[0].wait_recv()
        acc = jnp.dot(from_right[...], w, preferred_element_type=jnp.float32)
        out_ref[pl.ds(right * m_per, m_per), :] = jnp.maximum(acc, 0.0)

        diag_pos = (my_pos + 2) % N_DEV
        r2r[0].wait_recv()
        r2r[1].wait_recv()
        acc = jnp.dot(diag[pl.ds(0, half)], w, preferred_element_type=jnp.float32)
        out_ref[pl.ds(diag_pos * m_per, half), :] = jnp.maximum(acc, 0.0)

        r2l[0].wait_recv()
        r2l[1].wait_recv()
        acc = jnp.dot(diag[pl.ds(half, half)], w, preferred_element_type=jnp.float32)
        out_ref[pl.ds(diag_pos * m_per + half, half), :] = jnp.maximum(acc, 0.0)

        for d in r1r + r1l + r2r + r2l:
            d.wait_send()

    return pl.pallas_call(
        body,
        out_shape=jax.ShapeDtypeStruct((N_DEV * m_per, n_per), jnp.float32),
        in_specs=[
            pl.BlockSpec(memory_space=pltpu.VMEM),
            pl.BlockSpec(memory_space=pltpu.VMEM),
        ],
        out_specs=pl.BlockSpec(memory_space=pltpu.VMEM),
        scratch_shapes=[
            pltpu.VMEM((m_per, k), jnp.bfloat16),
            pltpu.VMEM((m_per, k), jnp.bfloat16),
            pltpu.VMEM((m_per, k), jnp.bfloat16),
            pltpu.SemaphoreType.DMA((N_SEG,)),
            pltpu.SemaphoreType.DMA((N_SEG,)),
            pltpu.SemaphoreType.DMA((N_SEG,)),
            pltpu.SemaphoreType.DMA((N_SEG,)),
            pltpu.SemaphoreType.DMA((2,)),
            pltpu.SemaphoreType.DMA((2,)),
            pltpu.SemaphoreType.DMA((2,)),
            pltpu.SemaphoreType.DMA((2,)),
        ],
        compiler_params=pltpu.CompilerParams(collective_id=0),
    )(x, w_mat)


# device time: 15698 ns/iter; 1.1659x vs baseline; 1.0683x over previous
import jax
import jax.numpy as jnp
from jax import lax
from jax.experimental import pallas as pl
from jax.experimental.pallas import tpu as pltpu

N_DEV = 4
N_SEG = 4


def kernel(x, w_mat):
    x = x.astype(jnp.bfloat16)
    w_mat = w_mat.astype(jnp.bfloat16)
    m_per, k = x.shape
    _, n_per = w_mat.shape
    seg = m_per // N_SEG
    half = m_per // 2

    def body(x_ref, w_ref, out_ref,
             from_left, from_right, diag,
             s_r1r, s_r1l, r_r1r, r_r1l,
             s_r2r, s_r2l, r_r2r, r_r2l):
        my_pos = lax.axis_index("i")
        left = (my_pos - 1) % N_DEV
        right = (my_pos + 1) % N_DEV

        barrier_sem = pltpu.get_barrier_semaphore()
        for nbr in [left, right]:
            pl.semaphore_signal(
                barrier_sem, inc=1,
                device_id=(nbr,), device_id_type=pl.DeviceIdType.MESH,
            )
        pl.semaphore_wait(barrier_sem, 2)

        def seg_copy(src, dst, s, send_sem, recv_sem, idx, dev):
            return pltpu.make_async_remote_copy(
                src_ref=src.at[pl.ds(s * seg, seg)],
                dst_ref=dst.at[pl.ds(s * seg, seg)],
                send_sem=send_sem.at[idx], recv_sem=recv_sem.at[idx],
                device_id=(dev,), device_id_type=pl.DeviceIdType.MESH,
            )

        r1r = [seg_copy(x_ref, from_left, s, s_r1r, r_r1r, s, right)
               for s in range(N_SEG)]
        r1l = [seg_copy(x_ref, from_right, s, s_r1l, r_r1l, s, left)
               for s in range(N_SEG)]
        for s in range(N_SEG):
            r1r[s].start()
            r1l[N_SEG - 1 - s].start()

        w = w_ref[...]
        acc = jnp.dot(x_ref[...], w, preferred_element_type=jnp.float32)
        out_ref[pl.ds(my_pos * m_per, m_per), :] = jnp.maximum(acc, 0.0)

        n_fwd = N_SEG // 2
        r2r = [seg_copy(from_left, diag, j, s_r2r, r_r2r, j, right)
               for j in range(n_fwd)]
        r2l = [seg_copy(from_right, diag, N_SEG - 1 - j, s_r2l, r_r2l, j, left)
               for j in range(n_fwd)]

        for j in range(n_fwd):
            r1r[j].wait_recv()
            r2r[j].start()
            r1l[N_SEG - 1 - j].wait_recv()
            r2l[j].start()

        for s in range(n_fwd, N_SEG):
            r1r[s].wait_recv()
        acc = jnp.dot(from_left[...], w, preferred_element_type=jnp.float32)
        out_ref[pl.ds(left * m_per, m_per), :] = jnp.maximum(acc, 0.0)

        for s in reversed(range(n_fwd)):
            r1l[s].wait_recv()
        acc = jnp.dot(from_right[...], w, preferred_element_type=jnp.float32)
        out_ref[pl.ds(right * m_per, m_per), :] = jnp.maximum(acc, 0.0)

        diag_pos = (my_pos + 2) % N_DEV
        for j in range(n_fwd):
            r2r[j].wait_recv()
        acc = jnp.dot(diag[pl.ds(0, half)], w, preferred_element_type=jnp.float32)
        out_ref[pl.ds(diag_pos * m_per, half), :] = jnp.maximum(acc, 0.0)

        for j in range(n_fwd):
            r2l[j].wait_recv()
        acc = jnp.dot(diag[pl.ds(half, half)], w, preferred_element_type=jnp.float32)
        out_ref[pl.ds(diag_pos * m_per + half, half), :] = jnp.maximum(acc, 0.0)

        for d in r1r + r1l + r2r + r2l:
            d.wait_send()

    return pl.pallas_call(
        body,
        out_shape=jax.ShapeDtypeStruct((N_DEV * m_per, n_per), jnp.float32),
        in_specs=[
            pl.BlockSpec(memory_space=pltpu.VMEM),
            pl.BlockSpec(memory_space=pltpu.VMEM),
        ],
        out_specs=pl.BlockSpec(memory_space=pltpu.VMEM),
        scratch_shapes=[
            pltpu.VMEM((m_per, k), jnp.bfloat16),
            pltpu.VMEM((m_per, k), jnp.bfloat16),
            pltpu.VMEM((m_per, k), jnp.bfloat16),
            pltpu.SemaphoreType.DMA((N_SEG,)),
            pltpu.SemaphoreType.DMA((N_SEG,)),
            pltpu.SemaphoreType.DMA((N_SEG,)),
            pltpu.SemaphoreType.DMA((N_SEG,)),
            pltpu.SemaphoreType.DMA((N_SEG // 2,)),
            pltpu.SemaphoreType.DMA((N_SEG // 2,)),
            pltpu.SemaphoreType.DMA((N_SEG // 2,)),
            pltpu.SemaphoreType.DMA((N_SEG // 2,)),
        ],
        compiler_params=pltpu.CompilerParams(collective_id=0),
    )(x, w_mat)
